# baseline (speedup 1.0000x reference)
import functools

import jax
import jax.numpy as jnp
from jax import lax
from jax.experimental import pallas as pl
from jax.experimental.pallas import tpu as pltpu
from jax.experimental.pallas import tpu_sc as plsc

_INFO = plsc.get_sparse_core_info()
_NC = _INFO.num_cores
_NS = _INFO.num_subcores
_NW = _NC * _NS
_CHUNK = 128
_L = 16


def _make_kernel(batch):
    assert batch % (_NW * _CHUNK) == 0
    b_per_w = batch // _NW
    n_chunks = b_per_w // _CHUNK
    mesh = plsc.VectorSubcoreMesh(core_axis_name="c", subcore_axis_name="s")

    idx2 = pltpu.VMEM((n_chunks, _CHUNK), jnp.int32)
    fbuf = pltpu.VMEM((n_chunks, _CHUNK), jnp.float32)

    @functools.partial(
        pl.kernel,
        out_type=jax.ShapeDtypeStruct((batch,), jnp.float32),
        mesh=mesh,
        scratch_types=[
            pltpu.VMEM((n_chunks, _CHUNK), jnp.int32),
            pltpu.VMEM((n_chunks, _CHUNK), jnp.int32),
            idx2, idx2, idx2, idx2,
            fbuf, fbuf, fbuf, fbuf,
            pltpu.VMEM((b_per_w,), jnp.float32),
            pltpu.SemaphoreType.DMA,
        ],
    )
    def mf_kernel(user_hbm, item_hbm, ut_hbm, it_hbm, out_hbm,
                  idx_u, idx_i, ux_i, uy_i, ix_i, iy_i,
                  ux_v, uy_v, ix_v, iy_v, out_v, sem):
        wid = lax.axis_index("s") * _NC + lax.axis_index("c")
        base = wid * b_per_w

        for k in range(n_chunks):
            pltpu.sync_copy(user_hbm.at[pl.ds(base + k * _CHUNK, _CHUNK)],
                            idx_u.at[k])
            pltpu.sync_copy(item_hbm.at[pl.ds(base + k * _CHUNK, _CHUNK)],
                            idx_i.at[k])

        for k in range(n_chunks):
            for o in range(0, _CHUNK, _L):
                s = pl.ds(o, _L)
                u = idx_u[k, s]
                du = u + u
                ux_i[k, s] = du
                uy_i[k, s] = du + 1
                i = idx_i[k, s]
                di = i + i
                ix_i[k, s] = di
                iy_i[k, s] = di + 1

        copies = []
        for k in range(n_chunks):
            for tab, src_i, dst_v in ((ut_hbm, ux_i, ux_v),
                                      (ut_hbm, uy_i, uy_v),
                                      (it_hbm, ix_i, ix_v),
                                      (it_hbm, iy_i, iy_v)):
                copies.append(pltpu.async_copy(tab.at[src_i.at[k]],
                                               dst_v.at[k], sem))
        for c in copies:
            c.wait()

        for k in range(n_chunks):
            for o in range(0, _CHUNK, _L):
                s = pl.ds(o, _L)
                out_v[pl.ds(k * _CHUNK + o, _L)] = (
                    ux_v[k, s] * ix_v[k, s] + uy_v[k, s] * iy_v[k, s])

        pltpu.sync_copy(out_v, out_hbm.at[pl.ds(base, b_per_w)])

    return mf_kernel


def kernel(user, item, user_table, item_table):
    batch = user.shape[0]
    k = _make_kernel(batch)
    return k(user.astype(jnp.int32), item.astype(jnp.int32),
             user_table.reshape(-1), item_table.reshape(-1))


# speedup vs baseline: 54.7412x; 54.7412x over previous
import functools

import jax
import jax.numpy as jnp
from jax import lax
from jax.experimental import pallas as pl
from jax.experimental.pallas import tpu as pltpu
from jax.experimental.pallas import tpu_sc as plsc

_INFO = plsc.get_sparse_core_info()
_NC = _INFO.num_cores
_NS = _INFO.num_subcores
_NW = _NC * _NS
_L = 16
_CHUNK = 128


def _make_phase1(batch, n_rows, tail_len):
    seg = (n_rows // (_NS * _CHUNK)) * _CHUNK
    tail_start = n_rows - tail_len
    b_per_t = batch // _NS
    n_chunks = b_per_t // _CHUNK
    mesh = plsc.VectorSubcoreMesh(core_axis_name="c", subcore_axis_name="s")
    col_out = jax.ShapeDtypeStruct((batch,), jnp.float32)

    @functools.partial(
        pl.kernel,
        out_type=(col_out, col_out, col_out, col_out),
        mesh=mesh,
        scratch_types=[
            pltpu.VMEM_SHARED((n_rows,), jnp.float32),
            pltpu.VMEM((seg,), jnp.float32),
            pltpu.VMEM((n_chunks, _CHUNK), jnp.int32),
            pltpu.VMEM((b_per_t,), jnp.float32),
            pltpu.SemaphoreType.DMA,
        ],
    )
    def phase1(user_hbm, item_hbm, ut_hbm, it_hbm, utt_hbm, itt_hbm,
               ux_hbm, uy_hbm, ix_hbm, iy_hbm,
               sp, vbuf, idx_v, g, sem):
        cid = lax.axis_index("c")
        sid = lax.axis_index("s")

        def run(idx_hbm, tab, tab_tail, ox_hbm, oy_hbm):
            base = sid * b_per_t
            for k in range(n_chunks):
                pltpu.sync_copy(idx_hbm.at[pl.ds(base + k * _CHUNK, _CHUNK)],
                                idx_v.at[k])

            for row, out_hbm in ((0, ox_hbm), (1, oy_hbm)):
                start = sid * seg
                pltpu.sync_copy(tab.at[row, pl.ds(start, seg)], vbuf)
                pltpu.sync_copy(vbuf, sp.at[pl.ds(start, seg)])

                @pl.when(sid == 0)
                def _():
                    pltpu.sync_copy(tab_tail.at[row, pl.ds(0, tail_len)],
                                    vbuf.at[pl.ds(0, tail_len)])
                    pltpu.sync_copy(vbuf.at[pl.ds(0, tail_len)],
                                    sp.at[pl.ds(tail_start, tail_len)])

                plsc.subcore_barrier()

                copies = []
                for k in range(n_chunks):
                    copies.append(pltpu.async_copy(
                        sp.at[idx_v.at[k]],
                        g.at[pl.ds(k * _CHUNK, _CHUNK)], sem))
                for c in copies:
                    c.wait()
                pltpu.sync_copy(g, out_hbm.at[pl.ds(base, b_per_t)])
                plsc.subcore_barrier()

        @pl.when(cid == 0)
        def _():
            run(user_hbm, ut_hbm, utt_hbm, ux_hbm, uy_hbm)

        @pl.when(cid == 1)
        def _():
            run(item_hbm, it_hbm, itt_hbm, ix_hbm, iy_hbm)

    return phase1


def _make_phase2(batch):
    b_per_w = batch // _NW
    mesh = plsc.VectorSubcoreMesh(core_axis_name="c", subcore_axis_name="s")
    col = pltpu.VMEM((b_per_w,), jnp.float32)

    @functools.partial(
        pl.kernel,
        out_type=jax.ShapeDtypeStruct((batch,), jnp.float32),
        mesh=mesh,
        scratch_types=[col, col, col, col, col],
    )
    def phase2(ux_hbm, uy_hbm, ix_hbm, iy_hbm, out_hbm,
               ux, uy, ixv, iyv, ov):
        wid = lax.axis_index("s") * _NC + lax.axis_index("c")
        base = wid * b_per_w
        s_all = pl.ds(base, b_per_w)
        pltpu.sync_copy(ux_hbm.at[s_all], ux)
        pltpu.sync_copy(uy_hbm.at[s_all], uy)
        pltpu.sync_copy(ix_hbm.at[s_all], ixv)
        pltpu.sync_copy(iy_hbm.at[s_all], iyv)
        for o in range(0, b_per_w, _L):
            s = pl.ds(o, _L)
            ov[s] = ux[s] * ixv[s] + uy[s] * iyv[s]
        pltpu.sync_copy(ov, out_hbm.at[s_all])

    return phase2


def kernel(user, item, user_table, item_table):
    batch = user.shape[0]
    n_rows = user_table.shape[0]
    tail_len = 5 * _CHUNK
    p1 = _make_phase1(batch, n_rows, tail_len)
    p2 = _make_phase2(batch)
    ux, uy, ixc, iyc = p1(user.astype(jnp.int32), item.astype(jnp.int32),
                          user_table.T, item_table.T,
                          user_table[n_rows - tail_len:].T,
                          item_table[n_rows - tail_len:].T)
    return p2(ux, uy, ixc, iyc)


# speedup vs baseline: 60.9109x; 1.1127x over previous
import functools

import jax
import jax.numpy as jnp
from jax import lax
from jax.experimental import pallas as pl
from jax.experimental.pallas import tpu as pltpu
from jax.experimental.pallas import tpu_sc as plsc

_INFO = plsc.get_sparse_core_info()
_NC = _INFO.num_cores
_NS = _INFO.num_subcores
_NW = _NC * _NS
_L = 16
_CHUNK = 128


def _make_phase1(batch, n_rows, tail_len):
    seg = (n_rows // (_NS * _CHUNK)) * _CHUNK
    tail_start = n_rows - tail_len
    b_per_t = batch // _NS
    n_chunks = b_per_t // _CHUNK
    mesh = plsc.VectorSubcoreMesh(core_axis_name="c", subcore_axis_name="s")
    col_out = jax.ShapeDtypeStruct((batch,), jnp.float32)

    @functools.partial(
        pl.kernel,
        out_type=(col_out, col_out, col_out, col_out),
        mesh=mesh,
        scratch_types=[
            pltpu.VMEM_SHARED((n_rows,), jnp.float32),
            pltpu.VMEM((seg,), jnp.float32),
            pltpu.VMEM((n_chunks, _CHUNK), jnp.int32),
            pltpu.VMEM((b_per_t,), jnp.float32),
            pltpu.SemaphoreType.DMA,
        ],
    )
    def phase1(user_hbm, item_hbm, ut_hbm, it_hbm, utt_hbm, itt_hbm,
               ux_hbm, uy_hbm, ix_hbm, iy_hbm,
               sp, vbuf, idx_v, g, sem):
        cid = lax.axis_index("c")
        sid = lax.axis_index("s")

        def run(idx_hbm, tab, tab_tail, ox_hbm, oy_hbm):
            base = sid * b_per_t
            for k in range(n_chunks):
                pltpu.sync_copy(idx_hbm.at[pl.ds(base + k * _CHUNK, _CHUNK)],
                                idx_v.at[k])

            for row, out_hbm in ((0, ox_hbm), (1, oy_hbm)):
                start = sid * seg
                pltpu.sync_copy(tab.at[row, pl.ds(start, seg)], vbuf)
                pltpu.sync_copy(vbuf, sp.at[pl.ds(start, seg)])

                @pl.when(sid == 0)
                def _():
                    pltpu.sync_copy(tab_tail.at[row, pl.ds(0, tail_len)],
                                    vbuf.at[pl.ds(0, tail_len)])
                    pltpu.sync_copy(vbuf.at[pl.ds(0, tail_len)],
                                    sp.at[pl.ds(tail_start, tail_len)])

                plsc.subcore_barrier()

                copies = []
                for k in range(n_chunks):
                    copies.append(pltpu.async_copy(
                        sp.at[idx_v.at[k]],
                        g.at[pl.ds(k * _CHUNK, _CHUNK)], sem))
                for c in copies:
                    c.wait()
                pltpu.sync_copy(g, out_hbm.at[pl.ds(base, b_per_t)])
                plsc.subcore_barrier()

        @pl.when(cid == 0)
        def _():
            run(user_hbm, ut_hbm, utt_hbm, ux_hbm, uy_hbm)

        @pl.when(cid == 1)
        def _():
            run(item_hbm, it_hbm, itt_hbm, ix_hbm, iy_hbm)

    return phase1


def _make_phase2(rows, cols):
    def body(ux, uy, ixv, iyv, o):
        o[...] = ux[...] * ixv[...] + uy[...] * iyv[...]

    return pl.pallas_call(
        body,
        out_shape=jax.ShapeDtypeStruct((rows, cols), jnp.float32),
    )


def kernel(user, item, user_table, item_table):
    batch = user.shape[0]
    n_rows = user_table.shape[0]
    tail_len = 5 * _CHUNK
    p1 = _make_phase1(batch, n_rows, tail_len)
    rows = batch // 128
    p2 = _make_phase2(rows, 128)
    ux, uy, ixc, iyc = p1(user.astype(jnp.int32), item.astype(jnp.int32),
                          user_table.T, item_table.T,
                          user_table[n_rows - tail_len:].T,
                          item_table[n_rows - tail_len:].T)
    out2d = p2(ux.reshape(rows, 128), uy.reshape(rows, 128),
               ixc.reshape(rows, 128), iyc.reshape(rows, 128))
    return out2d.reshape(batch)


# speedup vs baseline: 61.3959x; 1.0080x over previous
import functools

import jax
import jax.numpy as jnp
from jax import lax
from jax.experimental import pallas as pl
from jax.experimental.pallas import tpu as pltpu
from jax.experimental.pallas import tpu_sc as plsc

_INFO = plsc.get_sparse_core_info()
_NC = _INFO.num_cores
_NS = _INFO.num_subcores
_NW = _NC * _NS
_L = 16
_CHUNK = 128


def _make_phase1(batch, n_rows, tail_len):
    seg = (n_rows // (_NS * _CHUNK)) * _CHUNK
    tail_start = n_rows - tail_len
    b_per_t = batch // _NS
    n_chunks = b_per_t // _CHUNK
    mesh = plsc.VectorSubcoreMesh(core_axis_name="c", subcore_axis_name="s")

    @functools.partial(
        pl.kernel,
        out_type=jax.ShapeDtypeStruct((4 * batch,), jnp.float32),
        mesh=mesh,
        scratch_types=[
            pltpu.VMEM_SHARED((n_rows,), jnp.float32),
            pltpu.VMEM((seg,), jnp.float32),
            pltpu.VMEM((n_chunks, _CHUNK), jnp.int32),
            pltpu.VMEM((b_per_t,), jnp.float32),
            pltpu.SemaphoreType.DMA,
        ],
    )
    def phase1(idx_hbm, ut_hbm, it_hbm, utt_hbm, itt_hbm, out_hbm,
               sp, vbuf, idx_v, g, sem):
        cid = lax.axis_index("c")
        sid = lax.axis_index("s")
        base = cid * batch + sid * b_per_t
        for k in range(n_chunks):
            pltpu.sync_copy(idx_hbm.at[pl.ds(base + k * _CHUNK, _CHUNK)],
                            idx_v.at[k])

        for row in (0, 1):
            start = sid * seg

            @pl.when(cid == 0)
            def _():
                pltpu.sync_copy(ut_hbm.at[row, pl.ds(start, seg)], vbuf)

            @pl.when(cid == 1)
            def _():
                pltpu.sync_copy(it_hbm.at[row, pl.ds(start, seg)], vbuf)

            pltpu.sync_copy(vbuf, sp.at[pl.ds(start, seg)])

            @pl.when(jnp.logical_and(sid == 0, cid == 0))
            def _():
                pltpu.sync_copy(utt_hbm.at[row, pl.ds(0, tail_len)],
                                vbuf.at[pl.ds(0, tail_len)])

            @pl.when(jnp.logical_and(sid == 0, cid == 1))
            def _():
                pltpu.sync_copy(itt_hbm.at[row, pl.ds(0, tail_len)],
                                vbuf.at[pl.ds(0, tail_len)])

            @pl.when(sid == 0)
            def _():
                pltpu.sync_copy(vbuf.at[pl.ds(0, tail_len)],
                                sp.at[pl.ds(tail_start, tail_len)])

            plsc.subcore_barrier()

            copies = []
            for k in range(n_chunks):
                copies.append(pltpu.async_copy(sp.at[idx_v.at[k]],
                                               g.at[pl.ds(k * _CHUNK, _CHUNK)],
                                               sem))
            for c in copies:
                c.wait()
            obase = (2 * cid + row) * batch + sid * b_per_t
            pltpu.sync_copy(g, out_hbm.at[pl.ds(obase, b_per_t)])
            plsc.subcore_barrier()

    return phase1

def _make_phase2(rows, cols):
    def body(cols4, o):
        o[...] = (cols4[0] * cols4[2] + cols4[1] * cols4[3])

    return pl.pallas_call(
        body,
        out_shape=jax.ShapeDtypeStruct((rows, cols), jnp.float32),
    )


def kernel(user, item, user_table, item_table):
    batch = user.shape[0]
    n_rows = user_table.shape[0]
    tail_len = 5 * _CHUNK
    p1 = _make_phase1(batch, n_rows, tail_len)
    rows = batch // 128
    p2 = _make_phase2(rows, 128)
    cat_idx = jnp.concatenate([user.astype(jnp.int32),
                               item.astype(jnp.int32)])
    cols = p1(cat_idx, user_table.T, item_table.T,
              user_table[n_rows - tail_len:].T,
              item_table[n_rows - tail_len:].T)
    out2d = p2(cols.reshape(4, rows, 128))
    return out2d.reshape(batch)


# speedup vs baseline: 62.2479x; 1.0139x over previous
import functools

import jax
import jax.numpy as jnp
from jax import lax
from jax.experimental import pallas as pl
from jax.experimental.pallas import tpu as pltpu
from jax.experimental.pallas import tpu_sc as plsc

_INFO = plsc.get_sparse_core_info()
_NC = _INFO.num_cores
_NS = _INFO.num_subcores
_NW = _NC * _NS
_L = 16
_CHUNK = 128


def _make_phase1(batch, n_rows, tail_len):
    seg = (n_rows // (_NS * _CHUNK)) * _CHUNK
    half = seg // 2
    tail_start = n_rows - tail_len
    b_per_t = batch // _NS
    n_chunks = b_per_t // _CHUNK
    mesh = plsc.VectorSubcoreMesh(core_axis_name="c", subcore_axis_name="s")

    @functools.partial(
        pl.kernel,
        out_type=jax.ShapeDtypeStruct((4 * batch,), jnp.float32),
        mesh=mesh,
        scratch_types=[
            pltpu.VMEM_SHARED((n_rows,), jnp.float32),
            pltpu.VMEM((half,), jnp.float32),
            pltpu.VMEM((half,), jnp.float32),
            pltpu.VMEM((tail_len,), jnp.float32),
            pltpu.VMEM((n_chunks, _CHUNK), jnp.int32),
            pltpu.VMEM((b_per_t,), jnp.float32),
            pltpu.VMEM((b_per_t,), jnp.float32),
            pltpu.SemaphoreType.DMA,
            pltpu.SemaphoreType.DMA,
            pltpu.SemaphoreType.DMA,
            pltpu.SemaphoreType.DMA,
            pltpu.SemaphoreType.DMA,
            pltpu.SemaphoreType.DMA,
        ],
    )
    def phase1(idx_hbm, ut_hbm, it_hbm, utt_hbm, itt_hbm, out_hbm,
               sp, va, vb, vt, idx_v, g0, g1,
               sem_ha, sem_hb, sem_sa, sem_sb, sem_g, sem_w):
        cid = lax.axis_index("c")
        sid = lax.axis_index("s")
        base = cid * batch + sid * b_per_t
        start = sid * seg

        def read_half(row, buf, off, sem):
            @pl.when(cid == 0)
            def _():
                pltpu.async_copy(ut_hbm.at[row, pl.ds(start + off, half)],
                                 buf, sem)

            @pl.when(cid == 1)
            def _():
                pltpu.async_copy(it_hbm.at[row, pl.ds(start + off, half)],
                                 buf, sem)

            return pltpu.make_async_copy(
                ut_hbm.at[row, pl.ds(start + off, half)], buf, sem)

        for k in range(n_chunks):
            pltpu.sync_copy(idx_hbm.at[pl.ds(base + k * _CHUNK, _CHUNK)],
                            idx_v.at[k])

        ra = read_half(0, va, 0, sem_ha)
        rb = read_half(0, vb, half, sem_hb)
        ra.wait()
        sa = pltpu.async_copy(va, sp.at[pl.ds(start, half)], sem_sa)
        rb.wait()
        sb = pltpu.async_copy(vb, sp.at[pl.ds(start + half, half)], sem_sb)
        sa.wait()
        ra1 = read_half(1, va, 0, sem_ha)
        sb.wait()
        rb1 = read_half(1, vb, half, sem_hb)

        @pl.when(jnp.logical_and(sid == 0, cid == 0))
        def _():
            pltpu.sync_copy(utt_hbm.at[0, pl.ds(0, tail_len)], vt)

        @pl.when(jnp.logical_and(sid == 0, cid == 1))
        def _():
            pltpu.sync_copy(itt_hbm.at[0, pl.ds(0, tail_len)], vt)

        @pl.when(sid == 0)
        def _():
            pltpu.sync_copy(vt, sp.at[pl.ds(tail_start, tail_len)])

        plsc.subcore_barrier()

        g0_copies = [pltpu.async_copy(sp.at[idx_v.at[k]],
                                      g0.at[pl.ds(k * _CHUNK, _CHUNK)], sem_g)
                     for k in range(n_chunks)]
        for c in g0_copies:
            c.wait()
        w0 = pltpu.async_copy(
            g0, out_hbm.at[pl.ds(2 * cid * batch + sid * b_per_t, b_per_t)],
            sem_w)
        plsc.subcore_barrier()

        ra1.wait()
        sa1 = pltpu.async_copy(va, sp.at[pl.ds(start, half)], sem_sa)
        rb1.wait()
        sb1 = pltpu.async_copy(vb, sp.at[pl.ds(start + half, half)], sem_sb)
        sa1.wait()
        sb1.wait()

        @pl.when(jnp.logical_and(sid == 0, cid == 0))
        def _():
            pltpu.sync_copy(utt_hbm.at[1, pl.ds(0, tail_len)], vt)

        @pl.when(jnp.logical_and(sid == 0, cid == 1))
        def _():
            pltpu.sync_copy(itt_hbm.at[1, pl.ds(0, tail_len)], vt)

        @pl.when(sid == 0)
        def _():
            pltpu.sync_copy(vt, sp.at[pl.ds(tail_start, tail_len)])

        plsc.subcore_barrier()

        g1_copies = [pltpu.async_copy(sp.at[idx_v.at[k]],
                                      g1.at[pl.ds(k * _CHUNK, _CHUNK)], sem_g)
                     for k in range(n_chunks)]
        for c in g1_copies:
            c.wait()
        pltpu.sync_copy(
            g1, out_hbm.at[pl.ds((2 * cid + 1) * batch + sid * b_per_t,
                                 b_per_t)])
        w0.wait()

    return phase1

def _make_phase2(rows, cols):
    def body(cols4, o):
        o[...] = (cols4[0] * cols4[2] + cols4[1] * cols4[3])

    return pl.pallas_call(
        body,
        out_shape=jax.ShapeDtypeStruct((rows, cols), jnp.float32),
    )


def kernel(user, item, user_table, item_table):
    batch = user.shape[0]
    n_rows = user_table.shape[0]
    tail_len = 5 * _CHUNK
    p1 = _make_phase1(batch, n_rows, tail_len)
    rows = batch // 128
    p2 = _make_phase2(rows, 128)
    cat_idx = jnp.concatenate([user.astype(jnp.int32),
                               item.astype(jnp.int32)])
    cols = p1(cat_idx, user_table.T, item_table.T,
              user_table[n_rows - tail_len:].T,
              item_table[n_rows - tail_len:].T)
    out2d = p2(cols.reshape(4, rows, 128))
    return out2d.reshape(batch)
